# baseline (speedup 1.0000x reference)
import functools

import jax
import jax.numpy as jnp
from jax import lax
from jax.experimental import pallas as pl
from jax.experimental.pallas import tpu as pltpu
from jax.experimental.pallas import tpu_sc as plsc

VOCAB = 1000000
EMB = 64
B = 4096
L = 200

NC = 2
NS = 16
NW = NC * NS
TOTAL = B * L
PER_W = TOTAL // NW
CHUNK = 128
NCH = PER_W // CHUNK

_mesh = plsc.VectorSubcoreMesh(core_axis_name="c", subcore_axis_name="s")


@functools.partial(
    pl.kernel,
    mesh=_mesh,
    compiler_params=pltpu.CompilerParams(use_tc_tiling_on_sc=False),
    out_type=jax.ShapeDtypeStruct((TOTAL, EMB), jnp.float32),
    scratch_types=[
        pltpu.VMEM((NCH, CHUNK), jnp.int32),
        pltpu.VMEM((CHUNK, EMB), jnp.float32),
        pltpu.SemaphoreType.DMA,
    ],
)
def _sc_gather(table_hbm, idx_hbm, out_hbm, idx_v, rows_v, sem):
    wid = lax.axis_index("s") * NC + lax.axis_index("c")
    pltpu.sync_copy(idx_hbm.at[pl.ds(wid * NCH, NCH)], idx_v)

    def body(j, carry):
        pltpu.async_copy(table_hbm.at[idx_v.at[j]], rows_v, sem).wait()
        pltpu.sync_copy(rows_v, out_hbm.at[pl.ds(wid * PER_W + j * CHUNK, CHUNK)])
        return carry

    lax.fori_loop(0, NCH, body, 0)


def kernel(x, table):
    idx = x.reshape(TOTAL).astype(jnp.int32).reshape(NW * NCH, CHUNK)
    out = _sc_gather(table, idx)
    return out.reshape(B, L, EMB)


# speedup vs baseline: 1.1170x; 1.1170x over previous
import functools

import jax
import jax.numpy as jnp
from jax import lax
from jax.experimental import pallas as pl
from jax.experimental.pallas import tpu as pltpu
from jax.experimental.pallas import tpu_sc as plsc

VOCAB = 1000000
EMB = 64
B = 4096
L = 200

NC = 2
NS = 16
NW = NC * NS
TOTAL = B * L
PER_W = TOTAL // NW
CHUNK = 128
NCH = PER_W // CHUNK
NBUF = 8
NLAG = 4
NGRP = NCH // NBUF

_mesh = plsc.VectorSubcoreMesh(core_axis_name="c", subcore_axis_name="s")


@functools.partial(
    pl.kernel,
    mesh=_mesh,
    compiler_params=pltpu.CompilerParams(use_tc_tiling_on_sc=False),
    out_type=jax.ShapeDtypeStruct((TOTAL, EMB), jnp.float32),
    scratch_types=[
        pltpu.VMEM((NCH, CHUNK), jnp.int32),
        pltpu.VMEM((NBUF, CHUNK, EMB), jnp.float32),
        pltpu.SemaphoreType.DMA,
        pltpu.SemaphoreType.DMA,
    ],
)
def _sc_gather(table_hbm, idx_hbm, out_hbm, idx_v, rows_v, gsem, ssem):
    wid = lax.axis_index("s") * NC + lax.axis_index("c")
    pltpu.sync_copy(idx_hbm.at[pl.ds(wid * NCH, NCH)], idx_v)
    out_base = wid * PER_W

    def start_gather(j, slot):
        pltpu.async_copy(table_hbm.at[idx_v.at[j]], rows_v.at[slot], gsem)

    def wait_gather(slot):
        pltpu.make_async_copy(
            table_hbm.at[idx_v.at[0]], rows_v.at[slot], gsem).wait()

    def start_store(j, slot):
        pltpu.async_copy(
            rows_v.at[slot], out_hbm.at[pl.ds(out_base + j * CHUNK, CHUNK)],
            ssem)

    def wait_store(slot):
        pltpu.make_async_copy(
            rows_v.at[slot], out_hbm.at[pl.ds(out_base, CHUNK)], ssem).wait()

    for j in range(NBUF):
        start_gather(j, j)
        if j >= NLAG:
            wait_gather(j - NLAG)
            start_store(j - NLAG, j - NLAG)

    def body(g, carry):
        j0 = g * NBUF
        for b in range(NBUF):
            j = j0 + b
            wait_store(b)
            start_gather(j, b)
            sb = (b - NLAG) % NBUF
            wait_gather(sb)
            start_store(j - NLAG, sb)
        return carry

    lax.fori_loop(1, NGRP, body, 0)

    for k in range(NLAG):
        j = NCH - NLAG + k
        slot = j % NBUF
        wait_gather(slot)
        start_store(j, slot)
    for b in range(NBUF):
        wait_store(b)


def kernel(x, table):
    idx = x.reshape(TOTAL).astype(jnp.int32).reshape(NW * NCH, CHUNK)
    out = _sc_gather(table, idx)
    return out.reshape(B, L, EMB)


# speedup vs baseline: 1.7843x; 1.5973x over previous
import functools

import jax
import jax.numpy as jnp
from jax import lax
from jax.experimental import pallas as pl
from jax.experimental.pallas import tpu as pltpu
from jax.experimental.pallas import tpu_sc as plsc

VOCAB = 1000000
EMB = 64
B = 4096
L = 200

NC = 2
NS = 16
NW = NC * NS
BLK = B // NW
ROW = 128
NBUF = 3
NLAG = 2
_PRO = NBUF + NLAG

_mesh = plsc.VectorSubcoreMesh(core_axis_name="c", subcore_axis_name="s")

_DNUMS = lax.GatherDimensionNumbers(
    offset_dims=(), collapsed_slice_dims=(0,), start_index_map=(0,))


def _rot(v, idx):
    return lax.gather(v, idx[:, None], _DNUMS, (1,),
                      mode=lax.GatherScatterMode.PROMISE_IN_BOUNDS)


@functools.partial(
    pl.kernel,
    mesh=_mesh,
    compiler_params=pltpu.CompilerParams(use_tc_tiling_on_sc=False),
    out_type=jax.ShapeDtypeStruct((L, EMB // 8, NW, 8, BLK), jnp.float32),
    scratch_types=[
        pltpu.VMEM((L, BLK), jnp.int32),
        pltpu.VMEM((BLK, ROW), jnp.float32),
        pltpu.VMEM((BLK, ROW), jnp.float32),
        pltpu.VMEM((BLK, ROW), jnp.float32),
        pltpu.VMEM((EMB // 8, 8, BLK), jnp.float32),
        pltpu.VMEM((EMB // 8, 8, BLK), jnp.float32),
        pltpu.VMEM((EMB // 8, 8, BLK), jnp.float32),
        pltpu.SemaphoreType.DMA,
        pltpu.SemaphoreType.DMA,
    ],
)
def _sc_gather(t128_hbm, idxt_hbm, out_hbm, idx_v,
               rows_0, rows_1, rows_2, t_0, t_1, t_2, gsem, ssem):
    rows = (rows_0, rows_1, rows_2)
    ts = (t_0, t_1, t_2)
    wid = lax.axis_index("s") * NC + lax.axis_index("c")
    pltpu.sync_copy(idxt_hbm.at[:, pl.ds(wid * BLK, BLK)], idx_v)

    def start_gather(l, slot):
        pltpu.async_copy(t128_hbm.at[idx_v.at[l]], rows[slot], gsem)

    def wait_gather():
        pltpu.make_async_copy(
            t128_hbm.at[idx_v.at[0]], rows_0, gsem).wait()

    def start_store(l, slot):
        pltpu.async_copy(ts[slot], out_hbm.at[l, :, wid], ssem)

    def wait_store():
        pltpu.make_async_copy(t_0, out_hbm.at[0, :, wid], ssem).wait()

    lane = lax.iota(jnp.int32, 16)
    stages = []
    for k in (1, 2, 4, 8):
        stages.append((k,
                       lax.rem(lane + (16 - k), 16),
                       lax.rem(lane + k, 16),
                       (lane & k) != 0))

    def transpose_block(vecs):
        vecs = list(vecs)
        for k, idx_r, idx_l, hi_mask in stages:
            for i in range(16):
                if i & k:
                    continue
                p = i | k
                lo, hi = vecs[i], vecs[p]
                vecs[i] = jnp.where(hi_mask, _rot(hi, idx_r), lo)
                vecs[p] = jnp.where(hi_mask, hi, _rot(lo, idx_l))
        return vecs

    def transpose_tile(slot):
        src, dst = rows[slot], ts[slot]

        def ebody(e, carry):
            bb = lax.div(e, 4)
            cc = lax.rem(e, 4)
            b0 = bb * 16
            c0 = cc * 16
            vin = [src[b0 + i, pl.ds(c0, 16)] for i in range(16)]
            vout = transpose_block(vin)
            for i in range(16):
                d = c0 + i
                dst[lax.div(d, 8), lax.rem(d, 8), pl.ds(b0, 16)] = vout[i]
            return carry

        lax.fori_loop(0, (BLK // 16) * (EMB // 16), ebody, 0)

    def stage2(i, slot):
        wait_gather()
        transpose_tile(slot)
        start_store(i, slot)

    for l in range(_PRO):
        start_gather(l, l % NBUF)
        if l >= NLAG:
            stage2(l - NLAG, (l - NLAG) % NBUF)

    def body(g, carry):
        l0 = _PRO + g * NBUF
        for k in range(NBUF):
            l = l0 + k
            wait_store()
            start_gather(l, (_PRO + k) % NBUF)
            stage2(l - NLAG, (_PRO - NLAG + k) % NBUF)
        return carry

    lax.fori_loop(0, (L - _PRO) // NBUF, body, 0)

    for k in range(NLAG):
        i = L - NLAG + k
        wait_store()
        stage2(i, i % NBUF)
    for _ in range(NBUF):
        wait_store()


def kernel(x, table):
    t128 = jnp.pad(table, ((0, 0), (0, ROW - EMB)))
    idxt = x.T.astype(jnp.int32)
    out5 = _sc_gather(t128, idxt)
    return out5.transpose(2, 4, 0, 1, 3).reshape(B, L, EMB)


# speedup vs baseline: 1.7909x; 1.0037x over previous
import functools

import jax
import jax.numpy as jnp
from jax import lax
from jax.experimental import pallas as pl
from jax.experimental.pallas import tpu as pltpu
from jax.experimental.pallas import tpu_sc as plsc

VOCAB = 1000000
EMB = 64
B = 4096
L = 200

NC = 2
NS = 16
NW = NC * NS
BLK = B // NW
ROW = 128
NBUF = 3
NLAG = 2
_PRO = NBUF + NLAG

_mesh = plsc.VectorSubcoreMesh(core_axis_name="c", subcore_axis_name="s")

_DNUMS = lax.GatherDimensionNumbers(
    offset_dims=(), collapsed_slice_dims=(0,), start_index_map=(0,))


def _rot(v, idx):
    return lax.gather(v, idx[:, None], _DNUMS, (1,),
                      mode=lax.GatherScatterMode.PROMISE_IN_BOUNDS)


@functools.partial(
    pl.kernel,
    mesh=_mesh,
    compiler_params=pltpu.CompilerParams(use_tc_tiling_on_sc=False),
    out_type=jax.ShapeDtypeStruct((L, EMB // 8, NW, 8, BLK), jnp.float32),
    scratch_types=[
        pltpu.VMEM((L, BLK), jnp.int32),
        pltpu.VMEM((BLK, EMB), jnp.float32),
        pltpu.VMEM((BLK, EMB), jnp.float32),
        pltpu.VMEM((BLK, EMB), jnp.float32),
        pltpu.VMEM((EMB // 8, 8, BLK), jnp.float32),
        pltpu.VMEM((EMB // 8, 8, BLK), jnp.float32),
        pltpu.VMEM((EMB // 8, 8, BLK), jnp.float32),
        pltpu.SemaphoreType.DMA,
        pltpu.SemaphoreType.DMA,
    ],
)
def _sc_gather(t128_hbm, idxt_hbm, out_hbm, idx_v,
               rows_0, rows_1, rows_2, t_0, t_1, t_2, gsem, ssem):
    rows = (rows_0, rows_1, rows_2)
    ts = (t_0, t_1, t_2)
    wid = lax.axis_index("s") * NC + lax.axis_index("c")
    pltpu.sync_copy(idxt_hbm.at[:, pl.ds(wid * BLK, BLK)], idx_v)

    def start_gather(l, slot):
        pltpu.async_copy(t128_hbm.at[idx_v.at[l]], rows[slot], gsem)

    def wait_gather():
        pltpu.make_async_copy(
            t128_hbm.at[idx_v.at[0]], rows_0, gsem).wait()

    def start_store(l, slot):
        pltpu.async_copy(ts[slot], out_hbm.at[l, :, wid], ssem)

    def wait_store():
        pltpu.make_async_copy(t_0, out_hbm.at[0, :, wid], ssem).wait()

    lane = lax.iota(jnp.int32, 16)
    stages = []
    for k in (1, 2, 4, 8):
        stages.append((k,
                       lax.rem(lane + (16 - k), 16),
                       lax.rem(lane + k, 16),
                       (lane & k) != 0))

    def transpose_block(vecs):
        vecs = list(vecs)
        for k, idx_r, idx_l, hi_mask in stages:
            for i in range(16):
                if i & k:
                    continue
                p = i | k
                lo, hi = vecs[i], vecs[p]
                vecs[i] = jnp.where(hi_mask, _rot(hi, idx_r), lo)
                vecs[p] = jnp.where(hi_mask, hi, _rot(lo, idx_l))
        return vecs

    def transpose_tile(slot):
        src, dst = rows[slot], ts[slot]

        def ebody(e, carry):
            bb = lax.div(e, 4)
            cc = lax.rem(e, 4)
            b0 = bb * 16
            c0 = cc * 16
            vin = [src[b0 + i, pl.ds(c0, 16)] for i in range(16)]
            vout = transpose_block(vin)
            for i in range(16):
                d = c0 + i
                dst[lax.div(d, 8), lax.rem(d, 8), pl.ds(b0, 16)] = vout[i]
            return carry

        lax.fori_loop(0, (BLK // 16) * (EMB // 16), ebody, 0)

    def stage2(i, slot):
        wait_gather()
        transpose_tile(slot)
        start_store(i, slot)

    for l in range(_PRO):
        start_gather(l, l % NBUF)
        if l >= NLAG:
            stage2(l - NLAG, (l - NLAG) % NBUF)

    def body(g, carry):
        l0 = _PRO + g * NBUF
        for k in range(NBUF):
            l = l0 + k
            wait_store()
            start_gather(l, (_PRO + k) % NBUF)
            stage2(l - NLAG, (_PRO - NLAG + k) % NBUF)
        return carry

    lax.fori_loop(0, (L - _PRO) // NBUF, body, 0)

    for k in range(NLAG):
        i = L - NLAG + k
        wait_store()
        stage2(i, i % NBUF)
    for _ in range(NBUF):
        wait_store()


def kernel(x, table):
    t64 = jnp.pad(table, ((0, 0), (0, ROW - EMB))).reshape(2 * VOCAB, EMB)
    idxt = x.T.astype(jnp.int32) * 2
    out5 = _sc_gather(t64, idxt)
    return out5.transpose(2, 4, 0, 1, 3).reshape(B, L, EMB)
